# baseline (speedup 1.0000x reference)
import functools

import jax
import jax.numpy as jnp
from jax import lax
from jax.experimental import pallas as pl
from jax.experimental.pallas import tpu as pltpu
from jax.experimental.pallas import tpu_sc as plsc

_B = 16384
_LSEQ = 200
_V = 1000
_D = 16
_NOUT = 3

_NC = 2
_NS = 16
_NW = _NC * _NS
_RPW = _B // _NW
_CROWS = 128
_NCHUNK = _RPW // _CROWS
_CGROUPS = _CROWS // 16
_UNROLL = 8


def _tab_kernel(emb_ref, w_ref, b_ref, out_ref):
    t = lax.dot_general(
        w_ref[...], emb_ref[...], (((1,), (1,)), ((), ())),
        preferred_element_type=jnp.float32,
    )
    out_ref[...] = (t + b_ref[...]) * (1.0 / _LSEQ)


_sc_mesh = plsc.VectorSubcoreMesh(core_axis_name="c", subcore_axis_name="s")


@functools.partial(
    pl.kernel,
    mesh=_sc_mesh,
    out_type=jax.ShapeDtypeStruct((_B * _NOUT,), jnp.float32),
    scratch_types=[
        pltpu.VMEM((_NOUT, _V), jnp.float32),
        pltpu.VMEM((_CROWS, _LSEQ), jnp.int32),
        pltpu.VMEM((_CROWS, _LSEQ), jnp.int32),
        pltpu.VMEM((_RPW * _NOUT,), jnp.float32),
        pltpu.SemaphoreType.DMA,
        pltpu.SemaphoreType.DMA,
    ],
    compiler_params=pltpu.CompilerParams(needs_layout_passes=False),
)
def _sc_pool(x_hbm, tab_hbm, out_hbm, tab_v, x_v0, x_v1, out_v, sem0, sem1):
    wid = lax.axis_index("s") * _NC + lax.axis_index("c")
    row0 = wid * _RPW
    pltpu.sync_copy(tab_hbm, tab_v)

    bufs = (x_v0, x_v1)
    sems = (sem0, sem1)
    copies = [None, None]

    def start_chunk(c):
        b = c % 2
        copies[b] = pltpu.async_copy(
            x_hbm.at[pl.ds(row0 + c * _CROWS, _CROWS)], bufs[b], sems[b])

    start_chunk(0)

    lanes = lax.iota(jnp.int32, 16)
    j0 = jnp.zeros((16,), jnp.int32)
    j1 = jnp.full((16,), 1, dtype=jnp.int32)
    j2 = jnp.full((16,), 2, dtype=jnp.int32)
    zero = jnp.zeros((16,), jnp.float32)

    for c in range(_NCHUNK):
        copies[c % 2].wait()
        if c + 1 < _NCHUNK:
            start_chunk(c + 1)
        x_v = bufs[c % 2]

        def group_body(g, carry, x_v=x_v, c=c):
            rows = lanes + g * 16

            def l_body(l, accs):
                a0, a1, a2 = accs
                for k in range(_UNROLL):
                    col = jnp.full((16,), l * _UNROLL + k, dtype=jnp.int32)
                    xi = plsc.load_gather(x_v, [rows, col])
                    a0 = a0 + plsc.load_gather(tab_v, [j0, xi])
                    a1 = a1 + plsc.load_gather(tab_v, [j1, xi])
                    a2 = a2 + plsc.load_gather(tab_v, [j2, xi])
                return (a0, a1, a2)

            a0, a1, a2 = lax.fori_loop(
                0, _LSEQ // _UNROLL, l_body, (zero, zero, zero))
            obase = (lanes + (c * _CROWS + g * 16)) * _NOUT
            plsc.store_scatter(out_v, [obase], a0)
            plsc.store_scatter(out_v, [obase + 1], a1)
            plsc.store_scatter(out_v, [obase + 2], a2)
            return carry

        lax.fori_loop(0, _CGROUPS, group_body, 0)

    pltpu.sync_copy(out_v, out_hbm.at[pl.ds(row0 * _NOUT, _RPW * _NOUT)])


def kernel(x, emb_table, fc_w, fc_b):
    tab = pl.pallas_call(
        _tab_kernel,
        out_shape=jax.ShapeDtypeStruct((_NOUT, _V), jnp.float32),
    )(emb_table, fc_w, fc_b[:, None])
    out_flat = _sc_pool(x, tab)
    return out_flat.reshape(_B, _NOUT)


# speedup vs baseline: 1.4470x; 1.4470x over previous
import functools

import jax
import jax.numpy as jnp
from jax import lax
from jax.experimental import pallas as pl
from jax.experimental.pallas import tpu as pltpu
from jax.experimental.pallas import tpu_sc as plsc

_B = 16384
_LSEQ = 200
_V = 1000
_D = 16
_NOUT = 3

_NC = 2
_NS = 16
_NW = _NC * _NS
_RPW = _B // _NW
_CROWS = 128
_NCHUNK = _RPW // _CROWS
_NVEC = (_LSEQ + 15) // 16
_TW = _CROWS + 1
_UNROLL = 8


def _tab_kernel(emb_ref, w_ref, b_ref, out_ref):
    t = lax.dot_general(
        w_ref[...], emb_ref[...], (((1,), (1,)), ((), ())),
        preferred_element_type=jnp.float32,
    )
    out_ref[...] = (t + b_ref[...]) * (1.0 / _LSEQ)


_sc_mesh = plsc.VectorSubcoreMesh(core_axis_name="c", subcore_axis_name="s")


@functools.partial(
    pl.kernel,
    mesh=_sc_mesh,
    out_type=jax.ShapeDtypeStruct((_B * _NOUT,), jnp.float32),
    scratch_types=[
        pltpu.VMEM((_NOUT * _V,), jnp.float32),
        pltpu.VMEM((_CROWS, _LSEQ), jnp.int32),
        pltpu.VMEM((_CROWS, _LSEQ), jnp.int32),
        pltpu.VMEM((_NVEC * 16 * _TW,), jnp.int32),
        pltpu.VMEM((_RPW * _NOUT,), jnp.float32),
        pltpu.SemaphoreType.DMA,
        pltpu.SemaphoreType.DMA,
    ],
    compiler_params=pltpu.CompilerParams(needs_layout_passes=False),
)
def _sc_pool(x_hbm, tab_hbm, out_hbm, tab_v, x_v0, x_v1, xt_v, out_v,
             sem0, sem1):
    wid = lax.axis_index("s") * _NC + lax.axis_index("c")
    row0 = wid * _RPW
    pltpu.sync_copy(tab_hbm, tab_v)

    bufs = (x_v0, x_v1)
    sems = (sem0, sem1)
    copies = [None, None]

    def start_chunk(c):
        b = c % 2
        copies[b] = pltpu.async_copy(
            x_hbm.at[pl.ds(row0 + c * _CROWS, _CROWS)], bufs[b], sems[b])

    start_chunk(0)

    lanes = lax.iota(jnp.int32, 16)
    pat_t = lanes * _TW
    pat_o = lanes * _NOUT
    off1 = jnp.full((16,), _V, dtype=jnp.int32)
    off2 = jnp.full((16,), 2 * _V, dtype=jnp.int32)
    fzero = jnp.zeros((16,), jnp.float32)

    for c in range(_NCHUNK):
        copies[c % 2].wait()
        if c + 1 < _NCHUNK:
            start_chunk(c + 1)
        x_v = bufs[c % 2]

        @plsc.parallel_loop(0, _CROWS, unroll=2)
        def tr_body(r, x_v=x_v):
            for v in range(_NVEC):
                l0 = min(v * 16, _LSEQ - 16)
                d = x_v[r, pl.ds(l0, 16)]
                plsc.store_scatter(xt_v, [pat_t + (l0 * _TW + r)], d)

        def group_body(g, carry, c=c):
            gbase = g * 16

            @plsc.parallel_loop(0, _LSEQ, unroll=_UNROLL,
                                carry=(fzero, fzero, fzero))
            def l_body(l, accs):
                a0, a1, a2 = accs
                xi = xt_v[pl.ds(l * _TW + gbase, 16)]
                a0 = a0 + plsc.load_gather(tab_v, [xi])
                a1 = a1 + plsc.load_gather(tab_v, [xi + off1])
                a2 = a2 + plsc.load_gather(tab_v, [xi + off2])
                return (a0, a1, a2)

            a0, a1, a2 = l_body
            obase = (c * _CROWS + g * 16) * _NOUT
            plsc.store_scatter(out_v, [pat_o + obase], a0)
            plsc.store_scatter(out_v, [pat_o + (obase + 1)], a1)
            plsc.store_scatter(out_v, [pat_o + (obase + 2)], a2)
            return carry

        lax.fori_loop(0, _CROWS // 16, group_body, 0)

    pltpu.sync_copy(out_v, out_hbm.at[pl.ds(row0 * _NOUT, _RPW * _NOUT)])


def kernel(x, emb_table, fc_w, fc_b):
    tab = pl.pallas_call(
        _tab_kernel,
        out_shape=jax.ShapeDtypeStruct((_NOUT, _V), jnp.float32),
    )(emb_table, fc_w, fc_b[:, None])
    out_flat = _sc_pool(x, tab.reshape(_NOUT * _V))
    return out_flat.reshape(_B, _NOUT)


# speedup vs baseline: 1.8047x; 1.2471x over previous
import functools

import jax
import jax.numpy as jnp
from jax import lax
from jax.experimental import pallas as pl
from jax.experimental.pallas import tpu as pltpu
from jax.experimental.pallas import tpu_sc as plsc

_B = 16384
_LSEQ = 200
_V = 1000
_D = 16
_NOUT = 3

_NC = 2
_NS = 16
_NW = _NC * _NS
_RPW = _B // _NW
_CROWS = 128
_NCHUNK = _RPW // _CROWS
_NVEC = (_LSEQ + 15) // 16
_TW = _CROWS + 1
_UNROLL = 8


def _tab_kernel(emb_ref, w_ref, b_ref, out_ref):
    t = lax.dot_general(
        w_ref[...], emb_ref[...], (((1,), (1,)), ((), ())),
        preferred_element_type=jnp.float32,
    )
    out_ref[...] = (t + b_ref[...]) * (1.0 / _LSEQ)


_sc_mesh = plsc.VectorSubcoreMesh(core_axis_name="c", subcore_axis_name="s")


@functools.partial(
    pl.kernel,
    mesh=_sc_mesh,
    out_type=jax.ShapeDtypeStruct((_NOUT, _B), jnp.float32),
    scratch_types=[
        pltpu.VMEM((_NOUT * _V,), jnp.float32),
        pltpu.VMEM((_CROWS, _LSEQ), jnp.int32),
        pltpu.VMEM((_CROWS, _LSEQ), jnp.int32),
        pltpu.VMEM((_NVEC * 16 * _TW,), jnp.int32),
        pltpu.VMEM((_NOUT, _RPW), jnp.float32),
        pltpu.SemaphoreType.DMA,
        pltpu.SemaphoreType.DMA,
    ],
    compiler_params=pltpu.CompilerParams(needs_layout_passes=False),
)
def _sc_pool(x_hbm, tab_hbm, out_hbm, tab_v, x_v0, x_v1, xt_v, out_v,
             sem0, sem1):
    wid = lax.axis_index("s") * _NC + lax.axis_index("c")
    row0 = wid * _RPW
    pltpu.sync_copy(tab_hbm, tab_v)

    bufs = (x_v0, x_v1)
    sems = (sem0, sem1)
    copies = [None, None]

    def start_chunk(c):
        b = c % 2
        copies[b] = pltpu.async_copy(
            x_hbm.at[pl.ds(row0 + c * _CROWS, _CROWS)], bufs[b], sems[b])

    start_chunk(0)

    lanes = lax.iota(jnp.int32, 16)
    pat_t = lanes * _TW
    j0 = jnp.zeros((16,), dtype=jnp.int32)
    j1 = jnp.full((16,), 1, dtype=jnp.int32)
    j2 = jnp.full((16,), 2, dtype=jnp.int32)
    off1 = jnp.full((16,), _V, dtype=jnp.int32)
    off2 = jnp.full((16,), 2 * _V, dtype=jnp.int32)
    fzero = jnp.zeros((16,), jnp.float32)

    for c in range(_NCHUNK):
        copies[c % 2].wait()
        if c + 1 < _NCHUNK:
            start_chunk(c + 1)
        x_v = bufs[c % 2]

        @plsc.parallel_loop(0, _CROWS, unroll=2)
        def tr_body(r, x_v=x_v):
            for v in range(_NVEC):
                l0 = min(v * 16, _LSEQ - 16)
                d = x_v[r, pl.ds(l0, 16)]
                plsc.store_scatter(xt_v, [pat_t + (l0 * _TW + r)], d)

        def group_body(g, carry, c=c):
            gbase = g * 16

            @plsc.parallel_loop(0, _LSEQ, unroll=_UNROLL,
                                carry=(fzero, fzero, fzero))
            def l_body(l, accs):
                a0, a1, a2 = accs
                xi = xt_v[pl.ds(l * _TW + gbase, 16)]
                a0 = a0 + plsc.load_gather(tab_v, [xi])
                a1 = a1 + plsc.load_gather(tab_v, [xi + off1])
                a2 = a2 + plsc.load_gather(tab_v, [xi + off2])
                return (a0, a1, a2)

            a0, a1, a2 = l_body
            cols = lanes + (c * _CROWS + g * 16)
            plsc.store_scatter(out_v, [j0, cols], a0)
            plsc.store_scatter(out_v, [j1, cols], a1)
            plsc.store_scatter(out_v, [j2, cols], a2)
            return carry

        lax.fori_loop(0, _CROWS // 16, group_body, 0)

    pltpu.sync_copy(out_v, out_hbm.at[:, pl.ds(row0, _RPW)])


def kernel(x, emb_table, fc_w, fc_b):
    tab = pl.pallas_call(
        _tab_kernel,
        out_shape=jax.ShapeDtypeStruct((_NOUT, _V), jnp.float32),
    )(emb_table, fc_w, fc_b[:, None])
    out3 = _sc_pool(x, tab.reshape(_NOUT * _V))
    return out3.T


# speedup vs baseline: 1.9552x; 1.0834x over previous
import functools

import jax
import jax.numpy as jnp
from jax import lax
from jax.experimental import pallas as pl
from jax.experimental.pallas import tpu as pltpu
from jax.experimental.pallas import tpu_sc as plsc

_B = 16384
_LSEQ = 200
_V = 1000
_D = 16
_NOUT = 3

_NC = 2
_NS = 16
_NW = _NC * _NS
_RPW = _B // _NW
_CROWS = 128
_NCHUNK = _RPW // _CROWS
_NVEC = (_LSEQ + 15) // 16
_TW = _CROWS + 1
_UNROLL = 8


def _tab_kernel(emb_ref, w_ref, b_ref, out_ref):
    t = lax.dot_general(
        w_ref[...], emb_ref[...], (((1,), (1,)), ((), ())),
        preferred_element_type=jnp.float32,
    ) * (1.0 / _LSEQ) + b_ref[...] * (1.0 / _LSEQ)
    b0 = lax.bitcast_convert_type(
        t[0:1, :].astype(jnp.bfloat16), jnp.uint16).astype(jnp.uint32)
    b1 = lax.bitcast_convert_type(
        t[1:2, :].astype(jnp.bfloat16), jnp.uint16).astype(jnp.uint32)
    packed = lax.bitcast_convert_type(b0 | (b1 << 16), jnp.float32)
    out_ref[0:1, :] = packed
    out_ref[1:2, :] = t[2:3, :]


_sc_mesh = plsc.VectorSubcoreMesh(core_axis_name="c", subcore_axis_name="s")


@functools.partial(
    pl.kernel,
    mesh=_sc_mesh,
    out_type=jax.ShapeDtypeStruct((_NOUT, _B), jnp.float32),
    scratch_types=[
        pltpu.VMEM((2 * _V,), jnp.float32),
        pltpu.VMEM((_CROWS, _LSEQ), jnp.int32),
        pltpu.VMEM((_CROWS, _LSEQ), jnp.int32),
        pltpu.VMEM((_NVEC * 16 * _TW,), jnp.int32),
        pltpu.VMEM((_NOUT, _RPW), jnp.float32),
        pltpu.SemaphoreType.DMA,
        pltpu.SemaphoreType.DMA,
    ],
    compiler_params=pltpu.CompilerParams(needs_layout_passes=False),
)
def _sc_pool(x_hbm, tab_hbm, out_hbm, tab_v, x_v0, x_v1, xt_v, out_v,
             sem0, sem1):
    wid = lax.axis_index("s") * _NC + lax.axis_index("c")
    row0 = wid * _RPW
    pltpu.sync_copy(tab_hbm, tab_v)

    bufs = (x_v0, x_v1)
    sems = (sem0, sem1)
    copies = [None, None]

    def start_chunk(c):
        b = c % 2
        copies[b] = pltpu.async_copy(
            x_hbm.at[pl.ds(row0 + c * _CROWS, _CROWS)], bufs[b], sems[b])

    start_chunk(0)

    lanes = lax.iota(jnp.int32, 16)
    pat_t = lanes * _TW
    j0 = jnp.zeros((16,), dtype=jnp.int32)
    j1 = jnp.full((16,), 1, dtype=jnp.int32)
    j2 = jnp.full((16,), 2, dtype=jnp.int32)
    off2 = jnp.full((16,), _V, dtype=jnp.int32)
    himask = jnp.full((16,), -65536, dtype=jnp.int32)
    fzero = jnp.zeros((16,), jnp.float32)

    for c in range(_NCHUNK):
        copies[c % 2].wait()
        if c + 1 < _NCHUNK:
            start_chunk(c + 1)
        x_v = bufs[c % 2]

        @plsc.parallel_loop(0, _CROWS, unroll=2)
        def tr_body(r, x_v=x_v):
            for v in range(_NVEC):
                l0 = min(v * 16, _LSEQ - 16)
                d = x_v[r, pl.ds(l0, 16)]
                plsc.store_scatter(xt_v, [pat_t + (l0 * _TW + r)], d)

        def group_body(g, carry, c=c):
            gbase = g * 16

            @plsc.parallel_loop(0, _LSEQ, unroll=_UNROLL,
                                carry=(fzero, fzero, fzero))
            def l_body(l, accs):
                a0, a1, a2 = accs
                xi = xt_v[pl.ds(l * _TW + gbase, 16)]
                gp = plsc.bitcast(plsc.load_gather(tab_v, [xi]), jnp.int32)
                a0 = a0 + plsc.bitcast(gp << 16, jnp.float32)
                a1 = a1 + plsc.bitcast(gp & himask, jnp.float32)
                a2 = a2 + plsc.load_gather(tab_v, [xi + off2])
                return (a0, a1, a2)

            a0, a1, a2 = l_body
            cols = lanes + (c * _CROWS + g * 16)
            plsc.store_scatter(out_v, [j0, cols], a0)
            plsc.store_scatter(out_v, [j1, cols], a1)
            plsc.store_scatter(out_v, [j2, cols], a2)
            return carry

        lax.fori_loop(0, _CROWS // 16, group_body, 0)

    pltpu.sync_copy(out_v, out_hbm.at[:, pl.ds(row0, _RPW)])


def kernel(x, emb_table, fc_w, fc_b):
    tab = pl.pallas_call(
        _tab_kernel,
        out_shape=jax.ShapeDtypeStruct((2, _V), jnp.float32),
    )(emb_table, fc_w, fc_b[:, None])
    out3 = _sc_pool(x, tab.reshape(2 * _V))
    return out3.T


# speedup vs baseline: 1.9898x; 1.0177x over previous
import functools

import jax
import jax.numpy as jnp
from jax import lax
from jax.experimental import pallas as pl
from jax.experimental.pallas import tpu as pltpu
from jax.experimental.pallas import tpu_sc as plsc

_B = 16384
_LSEQ = 200
_V = 1000
_D = 16
_NOUT = 3

_NC = 2
_NS = 16
_NW = _NC * _NS
_RPW = _B // _NW
_CROWS = 128
_HROWS = _CROWS // 2
_NCHUNK = _RPW // _CROWS
_NVEC = (_LSEQ + 15) // 16
_TW = _HROWS + 1
_UNROLL = 8


def _tab_kernel(emb_ref, w_ref, b_ref, out_ref):
    t = lax.dot_general(
        w_ref[...], emb_ref[...], (((1,), (1,)), ((), ())),
        preferred_element_type=jnp.float32,
    ) * (1.0 / _LSEQ) + b_ref[...] * (1.0 / _LSEQ)
    b0 = lax.bitcast_convert_type(
        t[0:1, :].astype(jnp.bfloat16), jnp.uint16).astype(jnp.uint32)
    b1 = lax.bitcast_convert_type(
        t[1:2, :].astype(jnp.bfloat16), jnp.uint16).astype(jnp.uint32)
    packed = lax.bitcast_convert_type(b0 | (b1 << 16), jnp.float32)
    out_ref[0:1, :] = packed
    out_ref[1:2, :] = t[2:3, :]


_sc_mesh = plsc.VectorSubcoreMesh(core_axis_name="c", subcore_axis_name="s")


@functools.partial(
    pl.kernel,
    mesh=_sc_mesh,
    out_type=jax.ShapeDtypeStruct((_NOUT, _B), jnp.float32),
    scratch_types=[
        pltpu.VMEM((2 * _V,), jnp.float32),
        pltpu.VMEM((_CROWS, _LSEQ), jnp.int32),
        pltpu.VMEM((_CROWS, _LSEQ), jnp.int32),
        pltpu.VMEM((_NVEC * 16 * _TW,), jnp.int32),
        pltpu.VMEM((_NOUT, _RPW), jnp.float32),
        pltpu.SemaphoreType.DMA,
        pltpu.SemaphoreType.DMA,
    ],
    compiler_params=pltpu.CompilerParams(needs_layout_passes=False),
)
def _sc_pool(x_hbm, tab_hbm, out_hbm, tab_v, x_v0, x_v1, xt_v, out_v,
             sem0, sem1):
    wid = lax.axis_index("s") * _NC + lax.axis_index("c")
    row0 = wid * _RPW
    pltpu.sync_copy(tab_hbm, tab_v)

    bufs = (x_v0, x_v1)
    sems = (sem0, sem1)
    copies = [None, None]

    def start_chunk(c):
        b = c % 2
        copies[b] = pltpu.async_copy(
            x_hbm.at[pl.ds(row0 + c * _CROWS, _CROWS)], bufs[b], sems[b])

    start_chunk(0)

    lanes = lax.iota(jnp.int32, 16)
    pat_t = lanes * _TW
    off2 = jnp.full((16,), _V, dtype=jnp.int32)
    lomask = jnp.full((16,), 0xFFFF, dtype=jnp.int32)
    himask = jnp.full((16,), -65536, dtype=jnp.int32)
    fzero = jnp.zeros((16,), jnp.float32)
    zeros6 = (fzero,) * 6

    for c in range(_NCHUNK):
        copies[c % 2].wait()
        if c + 1 < _NCHUNK:
            start_chunk(c + 1)
        x_v = bufs[c % 2]

        @plsc.parallel_loop(0, _HROWS, unroll=2)
        def tr_body(r, x_v=x_v):
            for v in range(_NVEC):
                l0 = min(v * 16, _LSEQ - 16)
                dlo = x_v[r, pl.ds(l0, 16)]
                dhi = x_v[r + _HROWS, pl.ds(l0, 16)]
                plsc.store_scatter(
                    xt_v, [pat_t + (l0 * _TW + r)], dlo | (dhi << 16))

        def group_body(g, carry, c=c):
            gbase = g * 16

            @plsc.parallel_loop(0, _LSEQ, unroll=_UNROLL, carry=zeros6)
            def l_body(l, accs):
                a0l, a1l, a2l, a0h, a1h, a2h = accs
                xp = xt_v[pl.ds(l * _TW + gbase, 16)]
                xlo = xp & lomask
                xhi = lax.shift_right_logical(xp, 16)
                ge = plsc.bitcast(plsc.load_gather(tab_v, [xlo]), jnp.int32)
                go = plsc.bitcast(plsc.load_gather(tab_v, [xhi]), jnp.int32)
                a0l = a0l + plsc.bitcast(ge << 16, jnp.float32)
                a1l = a1l + plsc.bitcast(ge & himask, jnp.float32)
                a0h = a0h + plsc.bitcast(go << 16, jnp.float32)
                a1h = a1h + plsc.bitcast(go & himask, jnp.float32)
                a2l = a2l + plsc.load_gather(tab_v, [xlo + off2])
                a2h = a2h + plsc.load_gather(tab_v, [xhi + off2])
                return (a0l, a1l, a2l, a0h, a1h, a2h)

            a0l, a1l, a2l, a0h, a1h, a2h = l_body
            cb = c * _CROWS + gbase
            out_v[0, pl.ds(cb, 16)] = a0l
            out_v[1, pl.ds(cb, 16)] = a1l
            out_v[2, pl.ds(cb, 16)] = a2l
            out_v[0, pl.ds(cb + _HROWS, 16)] = a0h
            out_v[1, pl.ds(cb + _HROWS, 16)] = a1h
            out_v[2, pl.ds(cb + _HROWS, 16)] = a2h
            return carry

        lax.fori_loop(0, _HROWS // 16, group_body, 0)

    pltpu.sync_copy(out_v, out_hbm.at[:, pl.ds(row0, _RPW)])


def kernel(x, emb_table, fc_w, fc_b):
    tab = pl.pallas_call(
        _tab_kernel,
        out_shape=jax.ShapeDtypeStruct((2, _V), jnp.float32),
    )(emb_table, fc_w, fc_b[:, None])
    out3 = _sc_pool(x, tab.reshape(2 * _V))
    return out3.T


# speedup vs baseline: 2.8062x; 1.4103x over previous
import functools

import jax
import jax.numpy as jnp
from jax import lax
from jax.experimental import pallas as pl
from jax.experimental.pallas import tpu as pltpu
from jax.experimental.pallas import tpu_sc as plsc

_B = 16384
_LSEQ = 200
_V = 1000
_D = 16
_NOUT = 3

_NC = 2
_NS = 16
_NW = _NC * _NS
_RPW = _B // _NW
_CSEQ = 40
_NCHUNK = _LSEQ // _CSEQ
_NGROUP = _RPW // 16
_UNROLL = 8


def _tab_kernel(emb_ref, w_ref, b_ref, out_ref):
    t = lax.dot_general(
        w_ref[...], emb_ref[...], (((1,), (1,)), ((), ())),
        preferred_element_type=jnp.float32,
    ) * (1.0 / _LSEQ) + b_ref[...] * (1.0 / _LSEQ)
    b0 = lax.bitcast_convert_type(
        t[0:1, :].astype(jnp.bfloat16), jnp.uint16).astype(jnp.uint32)
    b1 = lax.bitcast_convert_type(
        t[1:2, :].astype(jnp.bfloat16), jnp.uint16).astype(jnp.uint32)
    packed = lax.bitcast_convert_type(b0 | (b1 << 16), jnp.float32)
    out_ref[0:1, :] = packed
    out_ref[1:2, :] = t[2:3, :]


_sc_mesh = plsc.VectorSubcoreMesh(core_axis_name="c", subcore_axis_name="s")


@functools.partial(
    pl.kernel,
    mesh=_sc_mesh,
    out_type=jax.ShapeDtypeStruct((_NOUT, _B), jnp.float32),
    scratch_types=[
        pltpu.VMEM((2 * _V,), jnp.float32),
        pltpu.VMEM((_CSEQ, _RPW), jnp.int32),
        pltpu.VMEM((_CSEQ, _RPW), jnp.int32),
        pltpu.VMEM((_NOUT, _RPW), jnp.float32),
        pltpu.SemaphoreType.DMA,
        pltpu.SemaphoreType.DMA,
    ],
    compiler_params=pltpu.CompilerParams(needs_layout_passes=False),
)
def _sc_pool(xt_hbm, tab_hbm, out_hbm, tab_v, x_v0, x_v1, out_v, sem0, sem1):
    wid = lax.axis_index("s") * _NC + lax.axis_index("c")
    row0 = wid * _RPW
    pltpu.sync_copy(tab_hbm, tab_v)

    bufs = (x_v0, x_v1)
    sems = (sem0, sem1)
    copies = [None, None]

    def start_chunk(c):
        b = c % 2
        copies[b] = pltpu.async_copy(
            xt_hbm.at[pl.ds(c * _CSEQ, _CSEQ), pl.ds(row0, _RPW)],
            bufs[b], sems[b])

    start_chunk(0)

    off2 = jnp.full((16,), _V, dtype=jnp.int32)
    himask = jnp.full((16,), -65536, dtype=jnp.int32)
    fzero = jnp.zeros((16,), jnp.float32)

    for c in range(_NCHUNK):
        copies[c % 2].wait()
        if c + 1 < _NCHUNK:
            start_chunk(c + 1)
        x_v = bufs[c % 2]

        def group_body(g, carry, x_v=x_v, c=c):
            gbase = g * 16

            @plsc.parallel_loop(0, _CSEQ, unroll=_UNROLL,
                                carry=(fzero, fzero, fzero))
            def l_body(l, accs):
                a0, a1, a2 = accs
                xi = x_v[l, pl.ds(gbase, 16)]
                gp = plsc.bitcast(plsc.load_gather(tab_v, [xi]), jnp.int32)
                a0 = a0 + plsc.bitcast(gp << 16, jnp.float32)
                a1 = a1 + plsc.bitcast(gp & himask, jnp.float32)
                a2 = a2 + plsc.load_gather(tab_v, [xi + off2])
                return (a0, a1, a2)

            a0, a1, a2 = l_body
            if c == 0:
                out_v[0, pl.ds(gbase, 16)] = a0
                out_v[1, pl.ds(gbase, 16)] = a1
                out_v[2, pl.ds(gbase, 16)] = a2
            else:
                out_v[0, pl.ds(gbase, 16)] = out_v[0, pl.ds(gbase, 16)] + a0
                out_v[1, pl.ds(gbase, 16)] = out_v[1, pl.ds(gbase, 16)] + a1
                out_v[2, pl.ds(gbase, 16)] = out_v[2, pl.ds(gbase, 16)] + a2
            return carry

        lax.fori_loop(0, _NGROUP, group_body, 0)

    pltpu.sync_copy(out_v, out_hbm.at[:, pl.ds(row0, _RPW)])


def kernel(x, emb_table, fc_w, fc_b):
    tab = pl.pallas_call(
        _tab_kernel,
        out_shape=jax.ShapeDtypeStruct((2, _V), jnp.float32),
    )(emb_table, fc_w, fc_b[:, None])
    out3 = _sc_pool(x.T, tab.reshape(2 * _V))
    return out3.T
